# baseline (speedup 1.0000x reference)
import functools

import jax
import jax.numpy as jnp
import numpy as np
from jax import lax
from jax.experimental import pallas as pl
from jax.experimental.pallas import tpu as pltpu
from jax.experimental.pallas import tpu_sc as plsc

D_MODEL = 1024
N_HEADS = 16
N_POINTS = 8
HEAD_DIM = 64
H = 64
W = 64

QB = 512
NW = 32


def _make_out_perm():
    k = np.arange(N_HEADS * HEAD_DIM)
    r = k % HEAD_DIM
    slot, j = r // 16, r % 16
    origd = np.choose(slot, [j, 32 + j, 16 + j, 48 + j])
    return (k // HEAD_DIM) * HEAD_DIM + origd


_OUT_PERM = _make_out_perm()


def _proj_body(q_ref, box_ref, w_ref, b_ref, idx_ref, wgt_ref, *, lq, qb):
    bidx = pl.program_id(0) // (lq // qb)
    q = q_ref[...]
    proj = jnp.dot(q, w_ref[...], preferred_element_type=jnp.float32,
                   precision=lax.Precision.DEFAULT) + b_ref[...]
    ox = proj[:, 0:128]
    oy = proj[:, 128:256]
    logits = proj[:, 256:384]
    e = jnp.exp(logits)
    r128 = lax.broadcasted_iota(jnp.int32, (128, 128), 0)
    c128 = lax.broadcasted_iota(jnp.int32, (128, 128), 1)
    gmat = (r128 // N_POINTS == c128 // N_POINTS).astype(jnp.float32)
    gsum = jnp.dot(e, gmat, preferred_element_type=jnp.float32,
                   precision=lax.Precision.HIGHEST)
    attn = e / gsum

    cx = box_ref[:, 0:1]
    cy = box_ref[:, 1:2]
    bw = box_ref[:, 2:3]
    bh = box_ref[:, 3:4]
    ix = (cx + ox * bw * 0.5) * float(W) - 0.5
    iy = (cy + oy * bh * 0.5) * float(H) - 0.5
    ix0 = jnp.floor(ix)
    iy0 = jnp.floor(iy)
    fx1 = ix - ix0
    fx0 = 1.0 - fx1
    fy1 = iy - iy0
    fy0 = 1.0 - fy1
    ix1 = ix0 + 1.0
    iy1 = iy0 + 1.0

    hcol = lax.broadcasted_iota(jnp.int32, (qb, 128), 1) // N_POINTS
    base = (bidx * N_HEADS + hcol) * (H * W)

    mx0 = ((ix0 >= 0.0) & (ix0 <= float(W - 1))).astype(jnp.float32)
    mx1 = ((ix1 >= 0.0) & (ix1 <= float(W - 1))).astype(jnp.float32)
    my0 = ((iy0 >= 0.0) & (iy0 <= float(H - 1))).astype(jnp.float32)
    my1 = ((iy1 >= 0.0) & (iy1 <= float(H - 1))).astype(jnp.float32)
    w_l = fx0 * mx0 + fx1 * mx1 * (ix0 == -1.0).astype(jnp.float32)
    w_r = fx1 * mx1 * (ix0 >= 0.0).astype(jnp.float32)
    w_t = fy0 * my0 + fy1 * my1 * (iy0 == -1.0).astype(jnp.float32)
    w_b = fy1 * my1 * (iy0 >= 0.0).astype(jnp.float32)
    px = jnp.clip(ix0, 0.0, float(W - 1)).astype(jnp.int32)
    py = jnp.clip(iy0, 0.0, float(H - 1)).astype(jnp.int32)
    idx_ref[...] = base + py * W + px
    wgt_ref[:, 0:128] = attn * w_t * w_l
    wgt_ref[:, 128:256] = attn * w_t * w_r
    wgt_ref[:, 256:384] = attn * w_b * w_l
    wgt_ref[:, 384:512] = attn * w_b * w_r


def _proj_call(qf, boxes, wc, bc, lq):
    blq = qf.shape[0]
    grid = blq // QB
    return pl.pallas_call(
        functools.partial(_proj_body, lq=lq, qb=QB),
        grid=(grid,),
        in_specs=[
            pl.BlockSpec((QB, D_MODEL), lambda i: (i, 0)),
            pl.BlockSpec((QB, 4), lambda i: (i, 0)),
            pl.BlockSpec(wc.shape, lambda i: (0, 0)),
            pl.BlockSpec(bc.shape, lambda i: (0, 0)),
        ],
        out_specs=[
            pl.BlockSpec((QB, 128), lambda i: (i, 0)),
            pl.BlockSpec((QB, 512), lambda i: (i, 0)),
        ],
        out_shape=[
            jax.ShapeDtypeStruct((blq, 128), jnp.int32),
            jax.ShapeDtypeStruct((blq, 512), jnp.float32),
        ],
    )(qf, boxes, wc, bc)


HW = H * W


def _table_body(m_ref, o_ref):
    m = m_ref[0]
    for h01 in range(4):
        sl = m[:, h01 * HEAD_DIM:(h01 + 1) * HEAD_DIM]
        xb = lax.bitcast_convert_type(sl, jnp.int32)
        rne = xb + 0x7FFF + (lax.shift_right_logical(xb, 16) & 1)
        rne = lax.shift_right_logical(rne, 16)
        packed = rne[:, 0:32] | (rne[:, 32:64] << 16)
        o_ref[h01, :, 0:32] = packed
        o_ref[h01, :, 32:64] = pltpu.roll(packed, HW - 1, 0)
        o_ref[h01, :, 64:96] = pltpu.roll(packed, HW - W, 0)
        o_ref[h01, :, 96:128] = pltpu.roll(packed, HW - W - 1, 0)


def _table_call(memory, b):
    return pl.pallas_call(
        _table_body,
        grid=(b, N_HEADS // 4),
        in_specs=[pl.BlockSpec((1, HW, 4 * HEAD_DIM), lambda i, j: (i, 0, j))],
        out_specs=pl.BlockSpec((4, HW, 2 * HEAD_DIM),
                               lambda i, j: (i * (N_HEADS // 4) + j, 0, 0)),
        out_shape=jax.ShapeDtypeStruct((b * N_HEADS, HW, 2 * HEAD_DIM),
                                       jnp.int32),
    )(memory).reshape(b * N_HEADS * HW, 2 * HEAD_DIM)


def _sc_gather_body(table_hbm, idx_hbm, wgt_hbm, out_hbm,
                    idxa_v, wgt_v, rows0_v, rows1_v, out_v,
                    sem_s0, sem_s1, sem_s2, sem_s3,
                    sem_g0, sem_g1, sem_o0, sem_o1,
                    *, q_per_worker):
    wid = lax.axis_index("s") * 2 + lax.axis_index("c")
    q0 = wid * q_per_worker
    sem_s = [sem_s0, sem_s1, sem_s2, sem_s3]
    sem_g = [sem_g0, sem_g1]
    sem_o = [sem_o0, sem_o1]
    rows = [rows0_v, rows1_v]
    qlast = q_per_worker - 1

    def clampq(s):
        return q0 + jnp.minimum(s, qlast)

    def stage(s, slot):
        bq = clampq(s)
        pltpu.async_copy(idx_hbm.at[bq], idxa_v.at[slot], sem_s[slot])
        pltpu.async_copy(wgt_hbm.at[bq], wgt_v.at[slot], sem_s[slot])

    def stage_wait(slot):
        pltpu.make_async_copy(idx_hbm.at[0],
                              idxa_v.at[slot], sem_s[slot]).wait()
        pltpu.make_async_copy(wgt_hbm.at[0],
                              wgt_v.at[slot], sem_s[slot]).wait()

    def gather(slot, p2):
        pltpu.async_copy(table_hbm.at[idxa_v.at[slot]], rows[p2], sem_g[p2])

    def gather_wait(slot, p2):
        pltpu.make_async_copy(table_hbm.at[idxa_v.at[slot]],
                              rows[p2], sem_g[p2]).wait()

    def out_wait(p2):
        pltpu.make_async_copy(out_v.at[p2],
                              out_hbm.at[0], sem_o[p2]).wait()

    stage(0, 0)
    stage(1, 1)
    stage_wait(0)
    gather(0, 0)

    def outer(i, carry):
        for b in range(4):
            s = i * 4 + b
            p2 = b % 2
            nslot = (b + 1) % 4
            stage_wait(nslot)
            gather(nslot, 1 - p2)
            stage(s + 2, (b + 2) % 4)
            gather_wait(b, p2)

            @pl.when(s >= 2)
            def _():
                out_wait(p2)

            src = rows[p2]

            def hp_body(hp, carry2, *, slot=b, p2=p2, src=src):
                wv = [wgt_v[slot, pl.ds(c * 128 + hp * 16, 16)]
                      for c in range(4)]
                for h01 in range(2):
                    lane0 = h01 * 8
                    acc = [jnp.zeros((16,), jnp.float32) for _ in range(4)]
                    for p in range(N_POINTS):
                        r = (hp * 2 + h01) * N_POINTS + p
                        ln = lane0 + p
                        for c in range(4):
                            for g in range(2):
                                w32 = src[r, pl.ds(c * 32 + g * 16, 16)]
                                lo = lax.bitcast_convert_type(
                                    w32 << 16, jnp.float32)
                                hi = lax.bitcast_convert_type(
                                    w32, jnp.float32)
                                acc[2 * g] = acc[2 * g] + wv[c][ln] * lo
                                acc[2 * g + 1] = (acc[2 * g + 1]
                                                  + wv[c][ln] * hi)
                    for d in range(4):
                        out_v[p2, pl.ds((hp * 2 + h01) * 64 + d * 16, 16)] = acc[d]
                return carry2

            for hp_s in range(8):
                hp_body(hp_s, 0)
            pltpu.async_copy(out_v.at[p2], out_hbm.at[q0 + s], sem_o[p2])
        return carry

    lax.fori_loop(0, q_per_worker // 4, outer, 0)
    stage_wait((q_per_worker + 1) % 4)
    gather_wait(q_per_worker % 4, q_per_worker % 2)
    out_wait(0)
    out_wait(1)


def _sc_gather(table2, idx_in, wgt_in, blq):
    qpw = blq // NW
    mesh = plsc.VectorSubcoreMesh(core_axis_name="c", subcore_axis_name="s")
    kfn = functools.partial(
        pl.kernel,
        mesh=mesh,
        out_type=jax.ShapeDtypeStruct((blq, N_HEADS * HEAD_DIM), jnp.float32),
        scratch_types=[
            pltpu.VMEM((4, 128), jnp.int32),
            pltpu.VMEM((4, 512), jnp.float32),
            pltpu.VMEM((128, 2 * HEAD_DIM), jnp.int32),
            pltpu.VMEM((128, 2 * HEAD_DIM), jnp.int32),
            pltpu.VMEM((2, N_HEADS * HEAD_DIM), jnp.float32),
            pltpu.SemaphoreType.DMA,
            pltpu.SemaphoreType.DMA,
            pltpu.SemaphoreType.DMA,
            pltpu.SemaphoreType.DMA,
            pltpu.SemaphoreType.DMA,
            pltpu.SemaphoreType.DMA,
            pltpu.SemaphoreType.DMA,
            pltpu.SemaphoreType.DMA,
        ],
    )(functools.partial(_sc_gather_body, q_per_worker=qpw))
    return kfn(table2, idx_in, wgt_in)


def _out_body(y_ref, w_ref, b_ref, z_ref, o_ref):
    o_ref[...] = (jnp.dot(y_ref[...], w_ref[...],
                          preferred_element_type=jnp.float32,
                          precision=lax.Precision.DEFAULT)
                  + b_ref[...] + z_ref[...])


def _out_call(y, w_out, b_out2, z2):
    blq = y.shape[0]
    grid = blq // QB
    return pl.pallas_call(
        _out_body,
        grid=(grid,),
        in_specs=[
            pl.BlockSpec((QB, D_MODEL), lambda i: (i, 0)),
            pl.BlockSpec((D_MODEL, D_MODEL), lambda i: (0, 0)),
            pl.BlockSpec((1, D_MODEL), lambda i: (0, 0)),
            pl.BlockSpec((1, 1), lambda i: (0, 0)),
        ],
        out_specs=pl.BlockSpec((QB, D_MODEL), lambda i: (i, 0)),
        out_shape=jax.ShapeDtypeStruct((blq, D_MODEL), jnp.float32),
    )(y, w_out, b_out2, z2)


def kernel(query, memory, reference_boxes, w_off, b_off, w_attn, b_attn,
           w_out, b_out, spatial_shape):
    b, lq, c = query.shape
    blq = b * lq
    w_off4 = w_off.reshape(c, N_HEADS, N_POINTS, 2)
    wc = jnp.concatenate([
        w_off4[..., 0].reshape(c, N_HEADS * N_POINTS),
        w_off4[..., 1].reshape(c, N_HEADS * N_POINTS),
        w_attn,
    ], axis=1)
    b_off4 = b_off.reshape(N_HEADS, N_POINTS, 2)
    bc = jnp.concatenate([
        b_off4[..., 0].reshape(-1), b_off4[..., 1].reshape(-1), b_attn,
    ]).reshape(1, 3 * N_HEADS * N_POINTS)

    qf = query.reshape(blq, c)
    boxes = reference_boxes.reshape(blq, 4)
    idx, wgt = _proj_call(qf, boxes, wc, bc, lq)
    table2 = _table_call(memory, b)

    y = _sc_gather(table2, idx, wgt, blq)
    w_out_p = w_out[_OUT_PERM, :]
    zero = (jnp.sum(spatial_shape) - (H + W)).astype(jnp.float32).reshape(1, 1)
    out = _out_call(y, w_out_p, b_out.reshape(1, D_MODEL), zero)
    return out.reshape(b, lq, c)


# speedup vs baseline: 1.7376x; 1.7376x over previous
import functools

import jax
import jax.numpy as jnp
import numpy as np
from jax import lax
from jax.experimental import pallas as pl
from jax.experimental.pallas import tpu as pltpu
from jax.experimental.pallas import tpu_sc as plsc

D_MODEL = 1024
N_HEADS = 16
N_POINTS = 8
HEAD_DIM = 64
H = 64
W = 64

QB = 512
NW = 32


def _make_out_perm():
    k = np.arange(N_HEADS * HEAD_DIM)
    r = k % HEAD_DIM
    slot, j = r // 16, r % 16
    origd = np.choose(slot, [j, 32 + j, 16 + j, 48 + j])
    return (k // HEAD_DIM) * HEAD_DIM + origd


_OUT_PERM = _make_out_perm()


def _proj_body(q_ref, box_ref, w_ref, b_ref, idx_ref, wgt_ref, *, lq, qb):
    bidx = pl.program_id(0) // (lq // qb)
    q = q_ref[...]
    proj = jnp.dot(q, w_ref[...], preferred_element_type=jnp.float32,
                   precision=lax.Precision.DEFAULT) + b_ref[...]
    ox = proj[:, 0:128]
    oy = proj[:, 128:256]
    logits = proj[:, 256:384]
    e = jnp.exp(logits)
    r128 = lax.broadcasted_iota(jnp.int32, (128, 128), 0)
    c128 = lax.broadcasted_iota(jnp.int32, (128, 128), 1)
    gmat = (r128 // N_POINTS == c128 // N_POINTS).astype(jnp.float32)
    gsum = jnp.dot(e, gmat, preferred_element_type=jnp.float32,
                   precision=lax.Precision.HIGHEST)
    attn = e / gsum

    cx = box_ref[:, 0:1]
    cy = box_ref[:, 1:2]
    bw = box_ref[:, 2:3]
    bh = box_ref[:, 3:4]
    ix = (cx + ox * bw * 0.5) * float(W) - 0.5
    iy = (cy + oy * bh * 0.5) * float(H) - 0.5
    ix0 = jnp.floor(ix)
    iy0 = jnp.floor(iy)
    fx1 = ix - ix0
    fx0 = 1.0 - fx1
    fy1 = iy - iy0
    fy0 = 1.0 - fy1
    ix1 = ix0 + 1.0
    iy1 = iy0 + 1.0

    hcol = lax.broadcasted_iota(jnp.int32, (qb, 128), 1) // N_POINTS
    base = (bidx * N_HEADS + hcol) * (H * W)

    mx0 = ((ix0 >= 0.0) & (ix0 <= float(W - 1))).astype(jnp.float32)
    mx1 = ((ix1 >= 0.0) & (ix1 <= float(W - 1))).astype(jnp.float32)
    my0 = ((iy0 >= 0.0) & (iy0 <= float(H - 1))).astype(jnp.float32)
    my1 = ((iy1 >= 0.0) & (iy1 <= float(H - 1))).astype(jnp.float32)
    w_l = fx0 * mx0 + fx1 * mx1 * (ix0 == -1.0).astype(jnp.float32)
    w_r = fx1 * mx1 * (ix0 >= 0.0).astype(jnp.float32)
    w_t = fy0 * my0 + fy1 * my1 * (iy0 == -1.0).astype(jnp.float32)
    w_b = fy1 * my1 * (iy0 >= 0.0).astype(jnp.float32)
    px = jnp.clip(ix0, 0.0, float(W - 1)).astype(jnp.int32)
    py = jnp.clip(iy0, 0.0, float(H - 1)).astype(jnp.int32)
    idx_ref[...] = base + py * W + px
    wgt_ref[:, 0:128] = attn * w_t * w_l
    wgt_ref[:, 128:256] = attn * w_t * w_r
    wgt_ref[:, 256:384] = attn * w_b * w_l
    wgt_ref[:, 384:512] = attn * w_b * w_r


def _proj_call(qf, boxes, wc, bc, lq):
    blq = qf.shape[0]
    grid = blq // QB
    return pl.pallas_call(
        functools.partial(_proj_body, lq=lq, qb=QB),
        grid=(grid,),
        in_specs=[
            pl.BlockSpec((QB, D_MODEL), lambda i: (i, 0)),
            pl.BlockSpec((QB, 4), lambda i: (i, 0)),
            pl.BlockSpec(wc.shape, lambda i: (0, 0)),
            pl.BlockSpec(bc.shape, lambda i: (0, 0)),
        ],
        out_specs=[
            pl.BlockSpec((QB, 128), lambda i: (i, 0)),
            pl.BlockSpec((QB, 512), lambda i: (i, 0)),
        ],
        out_shape=[
            jax.ShapeDtypeStruct((blq, 128), jnp.int32),
            jax.ShapeDtypeStruct((blq, 512), jnp.float32),
        ],
    )(qf, boxes, wc, bc)


HW = H * W


def _table_body(m_ref, o_ref):
    m = m_ref[0]
    for h01 in range(4):
        sl = m[:, h01 * HEAD_DIM:(h01 + 1) * HEAD_DIM]
        xb = lax.bitcast_convert_type(sl, jnp.int32)
        rne = xb + 0x7FFF + (lax.shift_right_logical(xb, 16) & 1)
        rne = lax.shift_right_logical(rne, 16)
        packed = rne[:, 0:32] | (rne[:, 32:64] << 16)
        o_ref[h01, :, 0:32] = packed
        o_ref[h01, :, 32:64] = pltpu.roll(packed, HW - 1, 0)
        o_ref[h01, :, 64:96] = pltpu.roll(packed, HW - W, 0)
        o_ref[h01, :, 96:128] = pltpu.roll(packed, HW - W - 1, 0)


def _table_call(memory, b):
    return pl.pallas_call(
        _table_body,
        grid=(b, N_HEADS // 4),
        in_specs=[pl.BlockSpec((1, HW, 4 * HEAD_DIM), lambda i, j: (i, 0, j))],
        out_specs=pl.BlockSpec((4, HW, 2 * HEAD_DIM),
                               lambda i, j: (i * (N_HEADS // 4) + j, 0, 0)),
        out_shape=jax.ShapeDtypeStruct((b * N_HEADS, HW, 2 * HEAD_DIM),
                                       jnp.int32),
    )(memory).reshape(b * N_HEADS * HW, 2 * HEAD_DIM)


def _sc_gather_body(table_hbm, idx_hbm, wgt_hbm, out_hbm,
                    idxa_v, wgt_v, rows0_v, rows1_v, out_v,
                    sem_s0, sem_s1, sem_s2, sem_s3,
                    sem_g0, sem_g1, sem_o0, sem_o1,
                    *, q_per_worker):
    wid = lax.axis_index("s") * 2 + lax.axis_index("c")
    q0 = wid * q_per_worker
    sem_s = [sem_s0, sem_s1, sem_s2, sem_s3]
    sem_g = [sem_g0, sem_g1]
    sem_o = [sem_o0, sem_o1]
    rows = [rows0_v, rows1_v]
    qlast = q_per_worker - 1

    def clampq(s):
        return q0 + jnp.minimum(s, qlast)

    def stage(s, slot):
        bq = clampq(s)
        pltpu.async_copy(idx_hbm.at[bq], idxa_v.at[slot], sem_s[slot])
        pltpu.async_copy(wgt_hbm.at[bq], wgt_v.at[slot], sem_s[slot])

    def stage_wait(slot):
        pltpu.make_async_copy(idx_hbm.at[0],
                              idxa_v.at[slot], sem_s[slot]).wait()
        pltpu.make_async_copy(wgt_hbm.at[0],
                              wgt_v.at[slot], sem_s[slot]).wait()

    def gather(slot, p2):
        pltpu.async_copy(table_hbm.at[idxa_v.at[slot]], rows[p2], sem_g[p2])

    def gather_wait(slot, p2):
        pltpu.make_async_copy(table_hbm.at[idxa_v.at[slot]],
                              rows[p2], sem_g[p2]).wait()

    def out_wait(p2):
        pltpu.make_async_copy(out_v.at[p2],
                              out_hbm.at[0], sem_o[p2]).wait()

    stage(0, 0)
    stage(1, 1)
    stage_wait(0)
    gather(0, 0)

    def outer(i, carry):
        for b in range(4):
            s = i * 4 + b
            p2 = b % 2
            nslot = (b + 1) % 4
            stage_wait(nslot)
            gather(nslot, 1 - p2)
            stage(s + 2, (b + 2) % 4)
            gather_wait(b, p2)

            @pl.when(s >= 2)
            def _():
                out_wait(p2)

            src = rows[p2]

            def hp_body(hp, carry2, *, slot=b, p2=p2, src=src):
                wv = [wgt_v[slot, pl.ds(c * 128 + hp * 16, 16)]
                      for c in range(4)]
                for h01 in range(2):
                    lane0 = h01 * 8
                    acc = [jnp.zeros((16,), jnp.float32) for _ in range(4)]
                    for p in range(N_POINTS):
                        r = (hp * 2 + h01) * N_POINTS + p
                        ln = lane0 + p
                        for c in range(4):
                            for g in range(2):
                                w32 = src[r, pl.ds(c * 32 + g * 16, 16)]
                                lo = lax.bitcast_convert_type(
                                    w32 << 16, jnp.float32)
                                hi = lax.bitcast_convert_type(
                                    w32, jnp.float32)
                                acc[2 * g] = acc[2 * g] + wv[c][ln] * lo
                                acc[2 * g + 1] = (acc[2 * g + 1]
                                                  + wv[c][ln] * hi)
                    for d in range(4):
                        out_v[p2, pl.ds((hp * 2 + h01) * 64 + d * 16, 16)] = acc[d]
                return carry2

            lax.fori_loop(0, 8, hp_body, 0, unroll=2)
            pltpu.async_copy(out_v.at[p2], out_hbm.at[q0 + s], sem_o[p2])
        return carry

    lax.fori_loop(0, q_per_worker // 4, outer, 0)
    stage_wait((q_per_worker + 1) % 4)
    gather_wait(q_per_worker % 4, q_per_worker % 2)
    out_wait(0)
    out_wait(1)


def _sc_gather(table2, idx_in, wgt_in, blq):
    qpw = blq // NW
    mesh = plsc.VectorSubcoreMesh(core_axis_name="c", subcore_axis_name="s")
    kfn = functools.partial(
        pl.kernel,
        mesh=mesh,
        out_type=jax.ShapeDtypeStruct((blq, N_HEADS * HEAD_DIM), jnp.float32),
        scratch_types=[
            pltpu.VMEM((4, 128), jnp.int32),
            pltpu.VMEM((4, 512), jnp.float32),
            pltpu.VMEM((128, 2 * HEAD_DIM), jnp.int32),
            pltpu.VMEM((128, 2 * HEAD_DIM), jnp.int32),
            pltpu.VMEM((2, N_HEADS * HEAD_DIM), jnp.float32),
            pltpu.SemaphoreType.DMA,
            pltpu.SemaphoreType.DMA,
            pltpu.SemaphoreType.DMA,
            pltpu.SemaphoreType.DMA,
            pltpu.SemaphoreType.DMA,
            pltpu.SemaphoreType.DMA,
            pltpu.SemaphoreType.DMA,
            pltpu.SemaphoreType.DMA,
        ],
    )(functools.partial(_sc_gather_body, q_per_worker=qpw))
    return kfn(table2, idx_in, wgt_in)


def _out_body(y_ref, w_ref, b_ref, z_ref, o_ref):
    o_ref[...] = (jnp.dot(y_ref[...], w_ref[...],
                          preferred_element_type=jnp.float32,
                          precision=lax.Precision.DEFAULT)
                  + b_ref[...] + z_ref[...])


def _out_call(y, w_out, b_out2, z2):
    blq = y.shape[0]
    grid = blq // QB
    return pl.pallas_call(
        _out_body,
        grid=(grid,),
        in_specs=[
            pl.BlockSpec((QB, D_MODEL), lambda i: (i, 0)),
            pl.BlockSpec((D_MODEL, D_MODEL), lambda i: (0, 0)),
            pl.BlockSpec((1, D_MODEL), lambda i: (0, 0)),
            pl.BlockSpec((1, 1), lambda i: (0, 0)),
        ],
        out_specs=pl.BlockSpec((QB, D_MODEL), lambda i: (i, 0)),
        out_shape=jax.ShapeDtypeStruct((blq, D_MODEL), jnp.float32),
    )(y, w_out, b_out2, z2)


def kernel(query, memory, reference_boxes, w_off, b_off, w_attn, b_attn,
           w_out, b_out, spatial_shape):
    b, lq, c = query.shape
    blq = b * lq
    w_off4 = w_off.reshape(c, N_HEADS, N_POINTS, 2)
    wc = jnp.concatenate([
        w_off4[..., 0].reshape(c, N_HEADS * N_POINTS),
        w_off4[..., 1].reshape(c, N_HEADS * N_POINTS),
        w_attn,
    ], axis=1)
    b_off4 = b_off.reshape(N_HEADS, N_POINTS, 2)
    bc = jnp.concatenate([
        b_off4[..., 0].reshape(-1), b_off4[..., 1].reshape(-1), b_attn,
    ]).reshape(1, 3 * N_HEADS * N_POINTS)

    qf = query.reshape(blq, c)
    boxes = reference_boxes.reshape(blq, 4)
    idx, wgt = _proj_call(qf, boxes, wc, bc, lq)
    table2 = _table_call(memory, b)

    y = _sc_gather(table2, idx, wgt, blq)
    w_out_p = w_out[_OUT_PERM, :]
    zero = (jnp.sum(spatial_shape) - (H + W)).astype(jnp.float32).reshape(1, 1)
    out = _out_call(y, w_out_p, b_out.reshape(1, D_MODEL), zero)
    return out.reshape(b, lq, c)


# speedup vs baseline: 1.9656x; 1.1312x over previous
import functools

import jax
import jax.numpy as jnp
import numpy as np
from jax import lax
from jax.experimental import pallas as pl
from jax.experimental.pallas import tpu as pltpu
from jax.experimental.pallas import tpu_sc as plsc

D_MODEL = 1024
N_HEADS = 16
N_POINTS = 8
HEAD_DIM = 64
H = 64
W = 64

QB = 512
NW = 32


def _make_out_perm():
    k = np.arange(N_HEADS * HEAD_DIM)
    r = k % HEAD_DIM
    slot, j = r // 16, r % 16
    origd = np.choose(slot, [j, 32 + j, 16 + j, 48 + j])
    return (k // HEAD_DIM) * HEAD_DIM + origd


_OUT_PERM = _make_out_perm()


def _proj_body(q_ref, box_ref, w_ref, b_ref, idx_ref, wgt_ref, *, lq, qb):
    bidx = pl.program_id(0) // (lq // qb)
    q = q_ref[...]
    proj = jnp.dot(q, w_ref[...], preferred_element_type=jnp.float32,
                   precision=lax.Precision.DEFAULT) + b_ref[...]
    ox = proj[:, 0:128]
    oy = proj[:, 128:256]
    logits = proj[:, 256:384]
    e = jnp.exp(logits)
    r128 = lax.broadcasted_iota(jnp.int32, (128, 128), 0)
    c128 = lax.broadcasted_iota(jnp.int32, (128, 128), 1)
    gmat = (r128 // N_POINTS == c128 // N_POINTS).astype(jnp.float32)
    gsum = jnp.dot(e, gmat, preferred_element_type=jnp.float32,
                   precision=lax.Precision.HIGHEST)
    attn = e / gsum

    cx = box_ref[:, 0:1]
    cy = box_ref[:, 1:2]
    bw = box_ref[:, 2:3]
    bh = box_ref[:, 3:4]
    ix = (cx + ox * bw * 0.5) * float(W) - 0.5
    iy = (cy + oy * bh * 0.5) * float(H) - 0.5
    ix0 = jnp.floor(ix)
    iy0 = jnp.floor(iy)
    fx1 = ix - ix0
    fx0 = 1.0 - fx1
    fy1 = iy - iy0
    fy0 = 1.0 - fy1
    ix1 = ix0 + 1.0
    iy1 = iy0 + 1.0

    hcol = lax.broadcasted_iota(jnp.int32, (qb, 128), 1) // N_POINTS
    base = (bidx * N_HEADS + hcol) * (H * W)

    mx0 = ((ix0 >= 0.0) & (ix0 <= float(W - 1))).astype(jnp.float32)
    mx1 = ((ix1 >= 0.0) & (ix1 <= float(W - 1))).astype(jnp.float32)
    my0 = ((iy0 >= 0.0) & (iy0 <= float(H - 1))).astype(jnp.float32)
    my1 = ((iy1 >= 0.0) & (iy1 <= float(H - 1))).astype(jnp.float32)
    w_l = fx0 * mx0 + fx1 * mx1 * (ix0 == -1.0).astype(jnp.float32)
    w_r = fx1 * mx1 * (ix0 >= 0.0).astype(jnp.float32)
    w_t = fy0 * my0 + fy1 * my1 * (iy0 == -1.0).astype(jnp.float32)
    w_b = fy1 * my1 * (iy0 >= 0.0).astype(jnp.float32)
    px = jnp.clip(ix0, 0.0, float(W - 1)).astype(jnp.int32)
    py = jnp.clip(iy0, 0.0, float(H - 1)).astype(jnp.int32)
    idx_ref[...] = base + py * W + px
    wgt_ref[:, 0:128] = attn * w_t * w_l
    wgt_ref[:, 128:256] = attn * w_t * w_r
    wgt_ref[:, 256:384] = attn * w_b * w_l
    wgt_ref[:, 384:512] = attn * w_b * w_r


def _proj_call(qf, boxes, wc, bc, lq):
    blq = qf.shape[0]
    grid = blq // QB
    return pl.pallas_call(
        functools.partial(_proj_body, lq=lq, qb=QB),
        grid=(grid,),
        in_specs=[
            pl.BlockSpec((QB, D_MODEL), lambda i: (i, 0)),
            pl.BlockSpec((QB, 4), lambda i: (i, 0)),
            pl.BlockSpec(wc.shape, lambda i: (0, 0)),
            pl.BlockSpec(bc.shape, lambda i: (0, 0)),
        ],
        out_specs=[
            pl.BlockSpec((QB, 128), lambda i: (i, 0)),
            pl.BlockSpec((QB, 512), lambda i: (i, 0)),
        ],
        out_shape=[
            jax.ShapeDtypeStruct((blq, 128), jnp.int32),
            jax.ShapeDtypeStruct((blq, 512), jnp.float32),
        ],
    )(qf, boxes, wc, bc)


HW = H * W


def _table_body(m_ref, o_ref):
    m = m_ref[0]
    for h01 in range(4):
        sl = m[:, h01 * HEAD_DIM:(h01 + 1) * HEAD_DIM]
        xb = lax.bitcast_convert_type(sl, jnp.int32)
        rne = xb + 0x7FFF + (lax.shift_right_logical(xb, 16) & 1)
        rne = lax.shift_right_logical(rne, 16)
        packed = rne[:, 0:32] | (rne[:, 32:64] << 16)
        o_ref[h01, :, 0:32] = packed
        o_ref[h01, :, 32:64] = pltpu.roll(packed, HW - 1, 0)
        o_ref[h01, :, 64:96] = pltpu.roll(packed, HW - W, 0)
        o_ref[h01, :, 96:128] = pltpu.roll(packed, HW - W - 1, 0)


def _table_call(memory, b):
    return pl.pallas_call(
        _table_body,
        grid=(b, N_HEADS // 4),
        in_specs=[pl.BlockSpec((1, HW, 4 * HEAD_DIM), lambda i, j: (i, 0, j))],
        out_specs=pl.BlockSpec((4, HW, 2 * HEAD_DIM),
                               lambda i, j: (i * (N_HEADS // 4) + j, 0, 0)),
        out_shape=jax.ShapeDtypeStruct((b * N_HEADS, HW, 2 * HEAD_DIM),
                                       jnp.int32),
    )(memory).reshape(b * N_HEADS * HW, 2 * HEAD_DIM)


def _sc_gather_body(table_hbm, idx_hbm, wgt_hbm, out_hbm,
                    idxa_v, wgt_v, rows0_v, rows1_v, out_v,
                    sem_s0, sem_s1, sem_s2, sem_s3,
                    sem_g0, sem_g1, sem_o0, sem_o1,
                    *, q_per_worker):
    wid = lax.axis_index("s") * 2 + lax.axis_index("c")
    q0 = wid * q_per_worker
    sem_s = [sem_s0, sem_s1, sem_s2, sem_s3]
    sem_g = [sem_g0, sem_g1]
    sem_o = [sem_o0, sem_o1]
    rows = [rows0_v, rows1_v]
    qlast = q_per_worker - 1

    def clampq(s):
        return q0 + jnp.minimum(s, qlast)

    def stage(s, slot):
        bq = clampq(s)
        pltpu.async_copy(idx_hbm.at[bq], idxa_v.at[slot], sem_s[slot])
        pltpu.async_copy(wgt_hbm.at[bq], wgt_v.at[slot], sem_s[slot])

    def stage_wait(slot):
        pltpu.make_async_copy(idx_hbm.at[0],
                              idxa_v.at[slot], sem_s[slot]).wait()
        pltpu.make_async_copy(wgt_hbm.at[0],
                              wgt_v.at[slot], sem_s[slot]).wait()

    def gather(slot, p2):
        pltpu.async_copy(table_hbm.at[idxa_v.at[slot]], rows[p2], sem_g[p2])

    def gather_wait(slot, p2):
        pltpu.make_async_copy(table_hbm.at[idxa_v.at[slot]],
                              rows[p2], sem_g[p2]).wait()

    def out_wait(p2):
        pltpu.make_async_copy(out_v.at[p2],
                              out_hbm.at[0], sem_o[p2]).wait()

    stage(0, 0)
    stage(1, 1)
    stage_wait(0)
    gather(0, 0)

    def outer(i, carry):
        for b in range(4):
            s = i * 4 + b
            p2 = b % 2
            nslot = (b + 1) % 4
            stage_wait(nslot)
            gather(nslot, 1 - p2)
            stage(s + 2, (b + 2) % 4)
            gather_wait(b, p2)

            @pl.when(s >= 2)
            def _():
                out_wait(p2)

            src = rows[p2]

            def hp_body(hp, carry2, *, slot=b, p2=p2, src=src):
                wv = [wgt_v[slot, pl.ds(c * 128 + hp * 16, 16)]
                      for c in range(4)]
                for h01 in range(2):
                    lane0 = h01 * 8
                    acc = [jnp.zeros((16,), jnp.float32) for _ in range(4)]
                    for p in range(N_POINTS):
                        r = (hp * 2 + h01) * N_POINTS + p
                        ln = lane0 + p
                        for c in range(4):
                            for g in range(2):
                                w32 = src[r, pl.ds(c * 32 + g * 16, 16)]
                                lo = lax.bitcast_convert_type(
                                    w32 << 16, jnp.float32)
                                hi = lax.bitcast_convert_type(
                                    w32, jnp.float32)
                                acc[2 * g] = acc[2 * g] + wv[c][ln] * lo
                                acc[2 * g + 1] = (acc[2 * g + 1]
                                                  + wv[c][ln] * hi)
                    for d in range(4):
                        out_v[p2, pl.ds((hp * 2 + h01) * 64 + d * 16, 16)] = acc[d]
                return carry2

            lax.fori_loop(0, 8, hp_body, 0)
            pltpu.async_copy(out_v.at[p2], out_hbm.at[q0 + s], sem_o[p2])
        return carry

    lax.fori_loop(0, q_per_worker // 4, outer, 0)
    stage_wait((q_per_worker + 1) % 4)
    gather_wait(q_per_worker % 4, q_per_worker % 2)
    out_wait(0)
    out_wait(1)


def _sc_gather(table2, idx_in, wgt_in, blq):
    qpw = blq // NW
    mesh = plsc.VectorSubcoreMesh(core_axis_name="c", subcore_axis_name="s")
    kfn = functools.partial(
        pl.kernel,
        mesh=mesh,
        out_type=jax.ShapeDtypeStruct((blq, N_HEADS * HEAD_DIM), jnp.float32),
        scratch_types=[
            pltpu.VMEM((4, 128), jnp.int32),
            pltpu.VMEM((4, 512), jnp.float32),
            pltpu.VMEM((128, 2 * HEAD_DIM), jnp.int32),
            pltpu.VMEM((128, 2 * HEAD_DIM), jnp.int32),
            pltpu.VMEM((2, N_HEADS * HEAD_DIM), jnp.float32),
            pltpu.SemaphoreType.DMA,
            pltpu.SemaphoreType.DMA,
            pltpu.SemaphoreType.DMA,
            pltpu.SemaphoreType.DMA,
            pltpu.SemaphoreType.DMA,
            pltpu.SemaphoreType.DMA,
            pltpu.SemaphoreType.DMA,
            pltpu.SemaphoreType.DMA,
        ],
    )(functools.partial(_sc_gather_body, q_per_worker=qpw))
    return kfn(table2, idx_in, wgt_in)


def _out_body(y_ref, w_ref, b_ref, z_ref, o_ref):
    o_ref[...] = (jnp.dot(y_ref[...], w_ref[...],
                          preferred_element_type=jnp.float32,
                          precision=lax.Precision.DEFAULT)
                  + b_ref[...] + z_ref[...])


def _out_call(y, w_out, b_out2, z2):
    blq = y.shape[0]
    grid = blq // QB
    return pl.pallas_call(
        _out_body,
        grid=(grid,),
        in_specs=[
            pl.BlockSpec((QB, D_MODEL), lambda i: (i, 0)),
            pl.BlockSpec((D_MODEL, D_MODEL), lambda i: (0, 0)),
            pl.BlockSpec((1, D_MODEL), lambda i: (0, 0)),
            pl.BlockSpec((1, 1), lambda i: (0, 0)),
        ],
        out_specs=pl.BlockSpec((QB, D_MODEL), lambda i: (i, 0)),
        out_shape=jax.ShapeDtypeStruct((blq, D_MODEL), jnp.float32),
    )(y, w_out, b_out2, z2)


def kernel(query, memory, reference_boxes, w_off, b_off, w_attn, b_attn,
           w_out, b_out, spatial_shape):
    b, lq, c = query.shape
    blq = b * lq
    w_off4 = w_off.reshape(c, N_HEADS, N_POINTS, 2)
    wc = jnp.concatenate([
        w_off4[..., 0].reshape(c, N_HEADS * N_POINTS),
        w_off4[..., 1].reshape(c, N_HEADS * N_POINTS),
        w_attn,
    ], axis=1)
    b_off4 = b_off.reshape(N_HEADS, N_POINTS, 2)
    bc = jnp.concatenate([
        b_off4[..., 0].reshape(-1), b_off4[..., 1].reshape(-1), b_attn,
    ]).reshape(1, 3 * N_HEADS * N_POINTS)

    qf = query.reshape(blq, c)
    boxes = reference_boxes.reshape(blq, 4)
    idx, wgt = _proj_call(qf, boxes, wc, bc, lq)
    table2 = _table_call(memory, b)

    y = _sc_gather(table2, idx, wgt, blq)
    w_out_p = w_out[_OUT_PERM, :]
    zero = (jnp.sum(spatial_shape) - (H + W)).astype(jnp.float32).reshape(1, 1)
    out = _out_call(y, w_out_p, b_out.reshape(1, D_MODEL), zero)
    return out.reshape(b, lq, c)
